# baseline (speedup 1.0000x reference)
import functools

import jax
import jax.numpy as jnp
from jax import lax
from jax.experimental import pallas as pl
from jax.experimental.pallas import tpu as pltpu
from jax.experimental.pallas import tpu_sc as plsc

NUM_TYPES = 10
PARAM_DIM = 4
ATOMS_PER_MOL = 20
N_ATOMS = 1_000_000
N_MOLS = N_ATOMS // ATOMS_PER_MOL

NC, NS, L = 2, 16, 16
NW = NC * NS

PAT = ATOMS_PER_MOL * PARAM_DIM
PARAMS_FLAT = N_ATOMS * PARAM_DIM
PUNIT = 25_000
P_UNITS_PER_W = PARAMS_FLAT // (PUNIT * NW)
PBUF = 25_040
PGROUPS = PBUF // PAT

MUNIT = 40_000
M_WORKERS = N_ATOMS // MUNIT
MGROUPS = MUNIT // PAT


def _sc_body(tp_hbm, ti_hbm, pout_hbm, mout_hbm,
             tp_v, ti_v, patt_v, pbuf_v, mbuf_v, psem, msem):
    w = lax.axis_index("s") * NC + lax.axis_index("c")

    pltpu.sync_copy(tp_hbm, tp_v)
    pltpu.sync_copy(ti_hbm, ti_v)

    iota = lax.iota(jnp.int32, L)
    for g in range(2 * PAT // L):
        j = iota + (g * L) % PAT
        a = j >> 2
        p = j & 3
        t = plsc.load_gather(ti_v, [a])
        vals = plsc.load_gather(tp_v, [t * PARAM_DIM + p])
        patt_v[pl.ds(g * L, L)] = vals

    phase = (w % 2) * (PUNIT % PAT)
    pvregs = [patt_v[pl.ds(phase + k * L, L)] for k in range(PAT // L)]

    def fill_params(t, _):
        for k in range(PAT // L):
            pbuf_v[pl.ds(t * PAT + k * L, L)] = pvregs[k]
        return 0

    lax.fori_loop(0, PGROUPS, fill_params, 0)

    phandles = []
    for t in range(P_UNITS_PER_W):
        off = (w + t * NW) * PUNIT
        phandles.append(
            pltpu.async_copy(pbuf_v.at[pl.ds(0, PUNIT)],
                             pout_hbm.at[pl.ds(off, PUNIT)], psem))

    base = [(iota + k * L) // ATOMS_PER_MOL for k in range(PAT // L)]

    @pl.when(w < M_WORKERS)
    def _mol():
        mol0 = w * (MUNIT // ATOMS_PER_MOL)

        def fill_mol(t, _):
            s = mol0 + t * (PAT // ATOMS_PER_MOL)
            for k in range(PAT // L):
                mbuf_v[pl.ds(t * PAT + k * L, L)] = base[k] + s
            return 0

        lax.fori_loop(0, MGROUPS, fill_mol, 0)
        pltpu.async_copy(mbuf_v, mout_hbm.at[pl.ds(w * MUNIT, MUNIT)],
                         msem).wait()

    for h in phandles:
        h.wait()


@jax.jit
def _sc_tile(tp_flat, ti_pad):
    mesh = plsc.VectorSubcoreMesh(core_axis_name="c", subcore_axis_name="s",
                                  num_cores=NC, num_subcores=NS)
    fn = pl.kernel(
        _sc_body,
        out_type=[jax.ShapeDtypeStruct((PARAMS_FLAT,), jnp.float32),
                  jax.ShapeDtypeStruct((N_ATOMS,), jnp.int32)],
        mesh=mesh,
        scratch_types=[
            pltpu.VMEM((2 * NUM_TYPES * PARAM_DIM,), jnp.float32),
            pltpu.VMEM((24,), jnp.int32),
            pltpu.VMEM((2 * PAT,), jnp.float32),
            pltpu.VMEM((PBUF,), jnp.float32),
            pltpu.VMEM((MUNIT,), jnp.int32),
            pltpu.SemaphoreType.DMA,
            pltpu.SemaphoreType.DMA,
        ],
        compiler_params=pltpu.CompilerParams(needs_layout_passes=False),
    )
    return fn(tp_flat, ti_pad)


def _copy_body(pos_ref, batch_ref, pos_out, batch_out):
    pos_out[...] = pos_ref[...]
    batch_out[...] = batch_ref[...]


@jax.jit
def _tc_passthrough(pos2d, batch2d):
    grid = 25
    rows = pos2d.shape[0] // grid
    return pl.pallas_call(
        _copy_body,
        grid=(grid,),
        in_specs=[pl.BlockSpec((rows, pos2d.shape[1]), lambda i: (i, 0)),
                  pl.BlockSpec((rows, batch2d.shape[1]), lambda i: (i, 0))],
        out_specs=[pl.BlockSpec((rows, pos2d.shape[1]), lambda i: (i, 0)),
                   pl.BlockSpec((rows, batch2d.shape[1]), lambda i: (i, 0))],
        out_shape=[jax.ShapeDtypeStruct(pos2d.shape, pos2d.dtype),
                   jax.ShapeDtypeStruct(batch2d.shape, batch2d.dtype)],
    )(pos2d, batch2d)


def kernel(pos, batch, type_params, type_index):
    tp_flat = jnp.pad(type_params.reshape(-1),
                      (0, NUM_TYPES * PARAM_DIM))
    ti_pad = jnp.pad(type_index, (0, 4))
    params_flat, sites_mol = _sc_tile(tp_flat, ti_pad)
    pos_c, batch_c = _tc_passthrough(pos.reshape(5000, 600),
                                     batch.reshape(5000, 200))
    sites_params = params_flat.reshape(N_ATOMS, PARAM_DIM)
    return (pos_c.reshape(N_ATOMS, 3), sites_params,
            batch_c.reshape(N_ATOMS), sites_mol)


# speedup vs baseline: 5.5457x; 5.5457x over previous
import functools

import jax
import jax.numpy as jnp
from jax import lax
from jax.experimental import pallas as pl
from jax.experimental.pallas import tpu as pltpu
from jax.experimental.pallas import tpu_sc as plsc

NUM_TYPES = 10
PARAM_DIM = 4
ATOMS_PER_MOL = 20
N_ATOMS = 1_000_000
N_MOLS = N_ATOMS // ATOMS_PER_MOL

NC, NS, L = 2, 16, 16
NW = NC * NS

PAT = ATOMS_PER_MOL * PARAM_DIM
PARAMS_FLAT = N_ATOMS * PARAM_DIM
PUNIT = 25_000
P_UNITS_PER_W = PARAMS_FLAT // (PUNIT * NW)
PBUF = 25_040
PGROUPS = PBUF // PAT

MUNIT = 40_000
M_WORKERS = N_ATOMS // MUNIT
MGROUPS = MUNIT // PAT


def _sc_body(tp_hbm, ti_hbm, pout_hbm, mout_hbm, one_hbm, zero_hbm,
             tp_v, ti_v, patt_v, pbuf_v, mbuf_v, one_v, zero_v, psem, msem):
    w = lax.axis_index("s") * NC + lax.axis_index("c")

    pltpu.sync_copy(tp_hbm, tp_v)
    pltpu.sync_copy(ti_hbm, ti_v)

    iota = lax.iota(jnp.int32, L)
    for g in range(2 * PAT // L):
        j = iota + (g * L) % PAT
        a = j >> 2
        p = j & 3
        t = plsc.load_gather(ti_v, [a])
        vals = plsc.load_gather(tp_v, [t * PARAM_DIM + p])
        patt_v[pl.ds(g * L, L)] = vals

    phase = (w % 2) * (PUNIT % PAT)
    pvregs = [patt_v[pl.ds(phase + k * L, L)] for k in range(PAT // L)]

    def fill_params(t, _):
        for k in range(PAT // L):
            pbuf_v[pl.ds(t * PAT + k * L, L)] = pvregs[k]
        return 0

    lax.fori_loop(0, PGROUPS, fill_params, 0)

    phandles = []
    for t in range(P_UNITS_PER_W):
        off = (w + t * NW) * PUNIT
        phandles.append(
            pltpu.async_copy(pbuf_v.at[pl.ds(0, PUNIT)],
                             pout_hbm.at[pl.ds(off, PUNIT)], psem))

    base = [(iota + k * L) // ATOMS_PER_MOL for k in range(PAT // L)]

    @pl.when(w < M_WORKERS)
    def _mol():
        mol0 = w * (MUNIT // ATOMS_PER_MOL)

        def fill_mol(t, _):
            s = mol0 + t * (PAT // ATOMS_PER_MOL)
            for k in range(PAT // L):
                mbuf_v[pl.ds(t * PAT + k * L, L)] = base[k] + s
            return 0

        lax.fori_loop(0, MGROUPS, fill_mol, 0)
        pltpu.async_copy(mbuf_v, mout_hbm.at[pl.ds(w * MUNIT, MUNIT)],
                         msem).wait()

    @pl.when(w == 0)
    def _consts():
        one_v[...] = jnp.full((L,), 1.0, jnp.float32)
        zero_v[...] = jnp.full((L,), 0, jnp.int32)
        pltpu.sync_copy(one_v, one_hbm)
        pltpu.sync_copy(zero_v, zero_hbm)

    for h in phandles:
        h.wait()


@jax.jit
def _sc_tile(tp_flat, ti_pad):
    mesh = plsc.VectorSubcoreMesh(core_axis_name="c", subcore_axis_name="s",
                                  num_cores=NC, num_subcores=NS)
    fn = pl.kernel(
        _sc_body,
        out_type=[jax.ShapeDtypeStruct((PARAMS_FLAT,), jnp.float32),
                  jax.ShapeDtypeStruct((N_ATOMS,), jnp.int32),
                  jax.ShapeDtypeStruct((L,), jnp.float32),
                  jax.ShapeDtypeStruct((L,), jnp.int32)],
        mesh=mesh,
        scratch_types=[
            pltpu.VMEM((2 * NUM_TYPES * PARAM_DIM,), jnp.float32),
            pltpu.VMEM((24,), jnp.int32),
            pltpu.VMEM((2 * PAT,), jnp.float32),
            pltpu.VMEM((PBUF,), jnp.float32),
            pltpu.VMEM((MUNIT,), jnp.int32),
            pltpu.VMEM((L,), jnp.float32),
            pltpu.VMEM((L,), jnp.int32),
            pltpu.SemaphoreType.DMA,
            pltpu.SemaphoreType.DMA,
        ],
        compiler_params=pltpu.CompilerParams(needs_layout_passes=False),
    )
    return fn(tp_flat, ti_pad)


def kernel(pos, batch, type_params, type_index):
    tp_flat = jnp.pad(type_params.reshape(-1),
                      (0, NUM_TYPES * PARAM_DIM))
    ti_pad = jnp.pad(type_index, (0, 4))
    params_flat, sites_mol, one, zero = _sc_tile(tp_flat, ti_pad)
    sites_params = params_flat.reshape(N_ATOMS, PARAM_DIM)
    sites_pos = pos * one[0]
    sites_batch = batch + zero[0]
    return (sites_pos, sites_params, sites_batch, sites_mol)


# speedup vs baseline: 64.2798x; 11.5909x over previous
import functools

import jax
import jax.numpy as jnp
from jax import lax
from jax.experimental import pallas as pl
from jax.experimental.pallas import tpu as pltpu
from jax.experimental.pallas import tpu_sc as plsc

NUM_TYPES = 10
PARAM_DIM = 4
ATOMS_PER_MOL = 20
N_ATOMS = 1_000_000
N_MOLS = N_ATOMS // ATOMS_PER_MOL

NC, NS, L = 2, 16, 16
NW = NC * NS

PAT = ATOMS_PER_MOL * PARAM_DIM
PARAMS_FLAT = N_ATOMS * PARAM_DIM
PUNIT = 25_000
P_UNITS_PER_W = PARAMS_FLAT // (PUNIT * NW)
PBUF = 25_040
PGROUPS = PBUF // PAT

MUNIT = 40_000
M_WORKERS = N_ATOMS // MUNIT
MGROUPS = MUNIT // PAT


def _sc_body(tp_hbm, ti_hbm, pout_hbm, mout_hbm, one_hbm, zero_hbm,
             tp_v, ti_v, pbuf_v, mbuf_v, one_v, zero_v, psem, msem):
    w = lax.axis_index("s") * NC + lax.axis_index("c")

    pltpu.sync_copy(tp_hbm, tp_v)
    pltpu.sync_copy(ti_hbm, ti_v)

    plane = w % PARAM_DIM
    sub = w // PARAM_DIM
    iota = lax.iota(jnp.int32, L)

    pvregs = []
    for k in range(PAT // L):
        m = (iota + (k * L) % ATOMS_PER_MOL) % ATOMS_PER_MOL
        t = plsc.load_gather(ti_v, [m])
        pvregs.append(plsc.load_gather(tp_v, [t * PARAM_DIM + plane]))

    def fill_params(t, _):
        for k in range(PAT // L):
            pbuf_v[pl.ds(t * PAT + k * L, L)] = pvregs[k]
        return 0

    lax.fori_loop(0, PGROUPS, fill_params, 0)

    phandles = []
    base_off = plane * N_ATOMS + sub * (N_ATOMS // (NW // PARAM_DIM))
    for t in range(P_UNITS_PER_W):
        phandles.append(
            pltpu.async_copy(pbuf_v.at[pl.ds(0, PUNIT)],
                             pout_hbm.at[pl.ds(base_off + t * PUNIT, PUNIT)],
                             psem))

    base = [(iota + k * L) // ATOMS_PER_MOL for k in range(PAT // L)]

    @pl.when(w < M_WORKERS)
    def _mol():
        mol0 = w * (MUNIT // ATOMS_PER_MOL)

        def fill_mol(t, _):
            s = mol0 + t * (PAT // ATOMS_PER_MOL)
            for k in range(PAT // L):
                mbuf_v[pl.ds(t * PAT + k * L, L)] = base[k] + s
            return 0

        lax.fori_loop(0, MGROUPS, fill_mol, 0)
        pltpu.async_copy(mbuf_v, mout_hbm.at[pl.ds(w * MUNIT, MUNIT)],
                         msem).wait()

    @pl.when(w == 0)
    def _consts():
        one_v[...] = jnp.full((L,), 1.0, jnp.float32)
        zero_v[...] = jnp.full((L,), 0, jnp.int32)
        pltpu.sync_copy(one_v, one_hbm)
        pltpu.sync_copy(zero_v, zero_hbm)

    for h in phandles:
        h.wait()


@jax.jit
def _sc_tile(tp_flat, ti_pad):
    mesh = plsc.VectorSubcoreMesh(core_axis_name="c", subcore_axis_name="s",
                                  num_cores=NC, num_subcores=NS)
    fn = pl.kernel(
        _sc_body,
        out_type=[jax.ShapeDtypeStruct((PARAMS_FLAT,), jnp.float32),
                  jax.ShapeDtypeStruct((N_ATOMS,), jnp.int32),
                  jax.ShapeDtypeStruct((L,), jnp.float32),
                  jax.ShapeDtypeStruct((L,), jnp.int32)],
        mesh=mesh,
        scratch_types=[
            pltpu.VMEM((2 * NUM_TYPES * PARAM_DIM,), jnp.float32),
            pltpu.VMEM((24,), jnp.int32),
            pltpu.VMEM((PBUF,), jnp.float32),
            pltpu.VMEM((MUNIT,), jnp.int32),
            pltpu.VMEM((L,), jnp.float32),
            pltpu.VMEM((L,), jnp.int32),
            pltpu.SemaphoreType.DMA,
            pltpu.SemaphoreType.DMA,
        ],
        compiler_params=pltpu.CompilerParams(needs_layout_passes=False),
    )
    return fn(tp_flat, ti_pad)


def kernel(pos, batch, type_params, type_index):
    tp_flat = jnp.pad(type_params.reshape(-1),
                      (0, NUM_TYPES * PARAM_DIM))
    ti_pad = jnp.pad(type_index, (0, 4))
    params_flat, sites_mol, one, zero = _sc_tile(tp_flat, ti_pad)
    sites_params = jnp.transpose(params_flat.reshape(PARAM_DIM, N_ATOMS))
    sites_pos = pos * one[0]
    sites_batch = batch + zero[0]
    return (sites_pos, sites_params, sites_batch, sites_mol)


# speedup vs baseline: 66.5256x; 1.0349x over previous
import functools

import jax
import jax.numpy as jnp
from jax import lax
from jax.experimental import pallas as pl
from jax.experimental.pallas import tpu as pltpu
from jax.experimental.pallas import tpu_sc as plsc

NUM_TYPES = 10
PARAM_DIM = 4
ATOMS_PER_MOL = 20
N_ATOMS = 1_000_000
N_MOLS = N_ATOMS // ATOMS_PER_MOL

NC, NS, L = 2, 16, 16
NW = NC * NS

PAT = ATOMS_PER_MOL * PARAM_DIM
PARAMS_FLAT = N_ATOMS * PARAM_DIM
PUNIT = 25_000
P_UNITS_PER_W = PARAMS_FLAT // (PUNIT * NW)
PBUF = 25_040
PGROUPS = PBUF // PAT

MUNIT = 40_000
M_WORKERS = N_ATOMS // MUNIT
MGROUPS = MUNIT // PAT


def _sc_body(tp_hbm, ti_hbm, pout_hbm, mout_hbm,
             tp_v, ti_v, pbuf_v, mbuf_v, psem, msem):
    w = lax.axis_index("s") * NC + lax.axis_index("c")

    pltpu.sync_copy(tp_hbm, tp_v)
    pltpu.sync_copy(ti_hbm, ti_v)

    plane = w % PARAM_DIM
    sub = w // PARAM_DIM
    iota = lax.iota(jnp.int32, L)

    pvregs = []
    for k in range(PAT // L):
        m = (iota + (k * L) % ATOMS_PER_MOL) % ATOMS_PER_MOL
        t = plsc.load_gather(ti_v, [m])
        pvregs.append(plsc.load_gather(tp_v, [t * PARAM_DIM + plane]))

    def fill_params(t, _):
        for k in range(PAT // L):
            pbuf_v[pl.ds(t * PAT + k * L, L)] = pvregs[k]
        return 0

    lax.fori_loop(0, PGROUPS, fill_params, 0)

    phandles = []
    base_off = plane * N_ATOMS + sub * (N_ATOMS // (NW // PARAM_DIM))
    for t in range(P_UNITS_PER_W):
        phandles.append(
            pltpu.async_copy(pbuf_v.at[pl.ds(0, PUNIT)],
                             pout_hbm.at[pl.ds(base_off + t * PUNIT, PUNIT)],
                             psem))

    base = [(iota + k * L) // ATOMS_PER_MOL for k in range(PAT // L)]

    @pl.when(w < M_WORKERS)
    def _mol():
        mol0 = w * (MUNIT // ATOMS_PER_MOL)

        def fill_mol(t, _):
            s = mol0 + t * (PAT // ATOMS_PER_MOL)
            for k in range(PAT // L):
                mbuf_v[pl.ds(t * PAT + k * L, L)] = base[k] + s
            return 0

        lax.fori_loop(0, MGROUPS, fill_mol, 0)
        pltpu.async_copy(mbuf_v, mout_hbm.at[pl.ds(w * MUNIT, MUNIT)],
                         msem).wait()

    for h in phandles:
        h.wait()


@jax.jit
def _sc_tile(tp_flat, ti_pad):
    mesh = plsc.VectorSubcoreMesh(core_axis_name="c", subcore_axis_name="s",
                                  num_cores=NC, num_subcores=NS)
    fn = pl.kernel(
        _sc_body,
        out_type=[jax.ShapeDtypeStruct((PARAMS_FLAT,), jnp.float32),
                  jax.ShapeDtypeStruct((N_ATOMS,), jnp.int32)],
        mesh=mesh,
        scratch_types=[
            pltpu.VMEM((2 * NUM_TYPES * PARAM_DIM,), jnp.float32),
            pltpu.VMEM((24,), jnp.int32),
            pltpu.VMEM((PBUF,), jnp.float32),
            pltpu.VMEM((MUNIT,), jnp.int32),
            pltpu.SemaphoreType.DMA,
            pltpu.SemaphoreType.DMA,
        ],
        compiler_params=pltpu.CompilerParams(needs_layout_passes=False),
    )
    return fn(tp_flat, ti_pad)


def kernel(pos, batch, type_params, type_index):
    tp_flat = jnp.pad(type_params.reshape(-1),
                      (0, NUM_TYPES * PARAM_DIM))
    ti_pad = jnp.pad(type_index, (0, 4))
    params_flat, sites_mol = _sc_tile(tp_flat, ti_pad)
    sites_params = jnp.transpose(params_flat.reshape(PARAM_DIM, N_ATOMS))
    one = type_params[0, 0] * 0.0 + 1.0
    zero = jnp.minimum(type_index[0], 0)
    sites_pos = pos * one
    sites_batch = batch + zero
    return (sites_pos, sites_params, sites_batch, sites_mol)


# speedup vs baseline: 67.2443x; 1.0108x over previous
import functools

import jax
import jax.numpy as jnp
from jax import lax
from jax.experimental import pallas as pl
from jax.experimental.pallas import tpu as pltpu
from jax.experimental.pallas import tpu_sc as plsc

NUM_TYPES = 10
PARAM_DIM = 4
ATOMS_PER_MOL = 20
N_ATOMS = 1_000_000
N_MOLS = N_ATOMS // ATOMS_PER_MOL

NC, NS, L = 2, 16, 16
NW = NC * NS

PAT = ATOMS_PER_MOL * PARAM_DIM
PARAMS_FLAT = N_ATOMS * PARAM_DIM
PUNIT = 25_000
P_UNITS_PER_W = PARAMS_FLAT // (PUNIT * NW)
PBUF = 25_040
PGROUPS = PBUF // PAT

MUNIT = 40_000
M_WORKERS = N_ATOMS // MUNIT
MGROUPS = MUNIT // PAT


def _sc_body(tp_hbm, ti_hbm, batch_hbm, pout_hbm, mout_hbm, bout_hbm,
             tp_v, ti_v, pbuf_v, mbuf_v, bbuf_v, psem, msem, bsem):
    w = lax.axis_index("s") * NC + lax.axis_index("c")

    @pl.when(w < M_WORKERS)
    def _batch_in():
        pltpu.async_copy(batch_hbm.at[pl.ds(w * MUNIT, MUNIT)],
                         bbuf_v, bsem)

    pltpu.sync_copy(tp_hbm, tp_v)
    pltpu.sync_copy(ti_hbm, ti_v)

    plane = w % PARAM_DIM
    sub = w // PARAM_DIM
    iota = lax.iota(jnp.int32, L)

    pvregs = []
    for k in range(PAT // L):
        m = (iota + (k * L) % ATOMS_PER_MOL) % ATOMS_PER_MOL
        t = plsc.load_gather(ti_v, [m])
        pvregs.append(plsc.load_gather(tp_v, [t * PARAM_DIM + plane]))

    def fill_params(t, _):
        for k in range(PAT // L):
            pbuf_v[pl.ds(t * PAT + k * L, L)] = pvregs[k]
        return 0

    lax.fori_loop(0, PGROUPS, fill_params, 0)

    phandles = []
    base_off = plane * N_ATOMS + sub * (N_ATOMS // (NW // PARAM_DIM))
    for t in range(P_UNITS_PER_W):
        phandles.append(
            pltpu.async_copy(pbuf_v.at[pl.ds(0, PUNIT)],
                             pout_hbm.at[pl.ds(base_off + t * PUNIT, PUNIT)],
                             psem))

    base = [(iota + k * L) // ATOMS_PER_MOL for k in range(PAT // L)]

    @pl.when(w < M_WORKERS)
    def _mol():
        mol0 = w * (MUNIT // ATOMS_PER_MOL)

        def fill_mol(t, _):
            s = mol0 + t * (PAT // ATOMS_PER_MOL)
            for k in range(PAT // L):
                mbuf_v[pl.ds(t * PAT + k * L, L)] = base[k] + s
            return 0

        lax.fori_loop(0, MGROUPS, fill_mol, 0)
        mh = pltpu.async_copy(mbuf_v, mout_hbm.at[pl.ds(w * MUNIT, MUNIT)],
                              msem)
        pltpu.make_async_copy(batch_hbm.at[pl.ds(w * MUNIT, MUNIT)],
                              bbuf_v, bsem).wait()
        pltpu.async_copy(bbuf_v, bout_hbm.at[pl.ds(w * MUNIT, MUNIT)],
                         bsem).wait()
        mh.wait()

    for h in phandles:
        h.wait()


@jax.jit
def _sc_tile(tp_flat, ti_pad, batch):
    mesh = plsc.VectorSubcoreMesh(core_axis_name="c", subcore_axis_name="s",
                                  num_cores=NC, num_subcores=NS)
    fn = pl.kernel(
        _sc_body,
        out_type=[jax.ShapeDtypeStruct((PARAMS_FLAT,), jnp.float32),
                  jax.ShapeDtypeStruct((N_ATOMS,), jnp.int32),
                  jax.ShapeDtypeStruct((N_ATOMS,), jnp.int32)],
        mesh=mesh,
        scratch_types=[
            pltpu.VMEM((2 * NUM_TYPES * PARAM_DIM,), jnp.float32),
            pltpu.VMEM((24,), jnp.int32),
            pltpu.VMEM((PBUF,), jnp.float32),
            pltpu.VMEM((MUNIT,), jnp.int32),
            pltpu.VMEM((MUNIT,), jnp.int32),
            pltpu.SemaphoreType.DMA,
            pltpu.SemaphoreType.DMA,
            pltpu.SemaphoreType.DMA,
        ],
        compiler_params=pltpu.CompilerParams(needs_layout_passes=False),
    )
    return fn(tp_flat, ti_pad, batch)


def kernel(pos, batch, type_params, type_index):
    tp_flat = jnp.pad(type_params.reshape(-1),
                      (0, NUM_TYPES * PARAM_DIM))
    ti_pad = jnp.pad(type_index, (0, 4))
    params_flat, sites_mol, sites_batch = _sc_tile(tp_flat, ti_pad, batch)
    sites_params = jnp.transpose(params_flat.reshape(PARAM_DIM, N_ATOMS))
    one = type_params[0, 0] * 0.0 + 1.0
    sites_pos = pos * one
    return (sites_pos, sites_params, sites_batch, sites_mol)


# speedup vs baseline: 67.5980x; 1.0053x over previous
import functools

import jax
import jax.numpy as jnp
from jax import lax
from jax.experimental import pallas as pl
from jax.experimental.pallas import tpu as pltpu
from jax.experimental.pallas import tpu_sc as plsc

NUM_TYPES = 10
PARAM_DIM = 4
ATOMS_PER_MOL = 20
N_ATOMS = 1_000_000
N_MOLS = N_ATOMS // ATOMS_PER_MOL

NC, NS, L = 2, 16, 16
NW = NC * NS

PAT = ATOMS_PER_MOL * PARAM_DIM
PARAMS_FLAT = N_ATOMS * PARAM_DIM
PUNIT = 25_000
P_UNITS_PER_W = PARAMS_FLAT // (PUNIT * NW)
PBUF = 25_040
PGROUPS = PBUF // PAT

MUNIT = 40_000
M_WORKERS = N_ATOMS // MUNIT
MGROUPS = MUNIT // PAT


def _sc_body(tp_hbm, ti_hbm, batch_hbm, pout_hbm, mout_hbm, bout_hbm,
             tp_v, ti_v, pbuf_v, mbuf_v, bbuf_v, psem, msem, bsem):
    w = lax.axis_index("s") * NC + lax.axis_index("c")

    @pl.when(w < M_WORKERS)
    def _batch_in():
        pltpu.async_copy(batch_hbm.at[pl.ds(w * MUNIT, MUNIT)],
                         bbuf_v, bsem)

    pltpu.sync_copy(tp_hbm, tp_v)
    pltpu.sync_copy(ti_hbm, ti_v)

    plane = w % PARAM_DIM
    sub = w // PARAM_DIM
    iota = lax.iota(jnp.int32, L)

    pvregs = []
    for k in range(PAT // L):
        m = (iota + (k * L) % ATOMS_PER_MOL) % ATOMS_PER_MOL
        t = plsc.load_gather(ti_v, [m])
        pvregs.append(plsc.load_gather(tp_v, [t * PARAM_DIM + plane]))

    def fill_params(t, _):
        for k in range(PAT // L):
            pbuf_v[pl.ds(t * PAT + k * L, L)] = pvregs[k]
        return 0

    lax.fori_loop(0, PGROUPS, fill_params, 0)

    phandles = []
    base_off = plane * N_ATOMS + sub * (N_ATOMS // (NW // PARAM_DIM))
    for t in range(P_UNITS_PER_W):
        phandles.append(
            pltpu.async_copy(pbuf_v.at[pl.ds(0, PUNIT)],
                             pout_hbm.at[pl.ds(base_off + t * PUNIT, PUNIT)],
                             psem))

    base = [(iota + k * L) // ATOMS_PER_MOL for k in range(PAT // L)]

    @pl.when(w < M_WORKERS)
    def _mol():
        mol0 = w * (MUNIT // ATOMS_PER_MOL)

        def fill_mol(t, _):
            s = mol0 + t * (PAT // ATOMS_PER_MOL)
            for k in range(PAT // L):
                mbuf_v[pl.ds(t * PAT + k * L, L)] = base[k] + s
            return 0

        lax.fori_loop(0, MGROUPS, fill_mol, 0)
        mh = pltpu.async_copy(mbuf_v, mout_hbm.at[pl.ds(w * MUNIT, MUNIT)],
                              msem)
        pltpu.make_async_copy(batch_hbm.at[pl.ds(w * MUNIT, MUNIT)],
                              bbuf_v, bsem).wait()
        pltpu.async_copy(bbuf_v, bout_hbm.at[pl.ds(w * MUNIT, MUNIT)],
                         bsem).wait()
        mh.wait()

    for h in phandles:
        h.wait()


@jax.jit
def _sc_tile(tp_flat, ti_pad, batch):
    mesh = plsc.VectorSubcoreMesh(core_axis_name="c", subcore_axis_name="s",
                                  num_cores=NC, num_subcores=NS)
    fn = pl.kernel(
        _sc_body,
        out_type=[jax.ShapeDtypeStruct((PARAMS_FLAT,), jnp.float32),
                  jax.ShapeDtypeStruct((N_ATOMS,), jnp.int32),
                  jax.ShapeDtypeStruct((N_ATOMS,), jnp.int32)],
        mesh=mesh,
        scratch_types=[
            pltpu.VMEM((NUM_TYPES * PARAM_DIM,), jnp.float32),
            pltpu.VMEM((ATOMS_PER_MOL,), jnp.int32),
            pltpu.VMEM((PBUF,), jnp.float32),
            pltpu.VMEM((MUNIT,), jnp.int32),
            pltpu.VMEM((MUNIT,), jnp.int32),
            pltpu.SemaphoreType.DMA,
            pltpu.SemaphoreType.DMA,
            pltpu.SemaphoreType.DMA,
        ],
        compiler_params=pltpu.CompilerParams(needs_layout_passes=False),
    )
    return fn(tp_flat, ti_pad, batch)


def kernel(pos, batch, type_params, type_index):
    tp_flat = type_params.reshape(-1)
    params_flat, sites_mol, sites_batch = _sc_tile(tp_flat, type_index,
                                                   batch)
    sites_params = jnp.transpose(params_flat.reshape(PARAM_DIM, N_ATOMS))
    one = (jnp.minimum(type_index[0], 0) + 1).astype(jnp.float32)
    sites_pos = pos * one
    return (sites_pos, sites_params, sites_batch, sites_mol)


# speedup vs baseline: 67.7008x; 1.0015x over previous
import jax
import jax.numpy as jnp
from jax import lax
from jax.experimental import pallas as pl
from jax.experimental.pallas import tpu as pltpu
from jax.experimental.pallas import tpu_sc as plsc

NUM_TYPES = 10
PARAM_DIM = 4
ATOMS_PER_MOL = 20
N_ATOMS = 1_000_000

NC, NS, L = 2, 16, 16
NW = NC * NS

PAT = ATOMS_PER_MOL * PARAM_DIM
PARAMS_FLAT = N_ATOMS * PARAM_DIM
PUNIT = 25_000
P_UNITS_PER_W = PARAMS_FLAT // (PUNIT * NW)
PBUF = 25_040
PGROUPS = PBUF // PAT

MUNIT = 40_000
M_WORKERS = N_ATOMS // MUNIT
MGROUPS = MUNIT // PAT


def _sc_body(tp_hbm, ti_hbm, batch_hbm, pout_hbm, mout_hbm, bout_hbm,
             tp_v, ti_v, pbuf_v, mbuf_v, bbuf_v, psem, msem, bsem):
    w = lax.axis_index("s") * NC + lax.axis_index("c")

    @pl.when(w < M_WORKERS)
    def _batch_in():
        pltpu.async_copy(batch_hbm.at[pl.ds(w * MUNIT, MUNIT)],
                         bbuf_v, bsem)

    pltpu.sync_copy(tp_hbm, tp_v)
    pltpu.sync_copy(ti_hbm, ti_v)

    plane = w % PARAM_DIM
    sub = w // PARAM_DIM
    iota = lax.iota(jnp.int32, L)

    pvregs = []
    for k in range(PAT // L):
        m = (iota + (k * L) % ATOMS_PER_MOL) % ATOMS_PER_MOL
        t = plsc.load_gather(ti_v, [m])
        pvregs.append(plsc.load_gather(tp_v, [t * PARAM_DIM + plane]))

    def fill_params(t, _):
        for k in range(PAT // L):
            pbuf_v[pl.ds(t * PAT + k * L, L)] = pvregs[k]
        return 0

    lax.fori_loop(0, PGROUPS, fill_params, 0)

    phandles = []
    base_off = plane * N_ATOMS + sub * (N_ATOMS // (NW // PARAM_DIM))
    for t in range(P_UNITS_PER_W):
        phandles.append(
            pltpu.async_copy(pbuf_v.at[pl.ds(0, PUNIT)],
                             pout_hbm.at[pl.ds(base_off + t * PUNIT, PUNIT)],
                             psem))

    base = [(iota + k * L) // ATOMS_PER_MOL for k in range(PAT // L)]

    @pl.when(w < M_WORKERS)
    def _mol():
        mol0 = w * (MUNIT // ATOMS_PER_MOL)

        def fill_mol(t, _):
            s = mol0 + t * (PAT // ATOMS_PER_MOL)
            for k in range(PAT // L):
                mbuf_v[pl.ds(t * PAT + k * L, L)] = base[k] + s
            return 0

        lax.fori_loop(0, MGROUPS, fill_mol, 0)
        mh = pltpu.async_copy(mbuf_v, mout_hbm.at[pl.ds(w * MUNIT, MUNIT)],
                              msem)
        pltpu.make_async_copy(batch_hbm.at[pl.ds(w * MUNIT, MUNIT)],
                              bbuf_v, bsem).wait()
        pltpu.async_copy(bbuf_v, bout_hbm.at[pl.ds(w * MUNIT, MUNIT)],
                         bsem).wait()
        mh.wait()

    for h in phandles:
        h.wait()


@jax.jit
def _sc_tile(tp_flat, type_index, batch):
    mesh = plsc.VectorSubcoreMesh(core_axis_name="c", subcore_axis_name="s",
                                  num_cores=NC, num_subcores=NS)
    fn = pl.kernel(
        _sc_body,
        out_type=[jax.ShapeDtypeStruct((PARAMS_FLAT,), jnp.float32),
                  jax.ShapeDtypeStruct((N_ATOMS,), jnp.int32),
                  jax.ShapeDtypeStruct((N_ATOMS,), jnp.int32)],
        mesh=mesh,
        scratch_types=[
            pltpu.VMEM((NUM_TYPES * PARAM_DIM,), jnp.float32),
            pltpu.VMEM((ATOMS_PER_MOL,), jnp.int32),
            pltpu.VMEM((PBUF,), jnp.float32),
            pltpu.VMEM((MUNIT,), jnp.int32),
            pltpu.VMEM((MUNIT,), jnp.int32),
            pltpu.SemaphoreType.DMA,
            pltpu.SemaphoreType.DMA,
            pltpu.SemaphoreType.DMA,
        ],
        compiler_params=pltpu.CompilerParams(needs_layout_passes=False),
    )
    return fn(tp_flat, type_index, batch)


def kernel(pos, batch, type_params, type_index):
    tp_flat = type_params.reshape(-1)
    params_flat, sites_mol, sites_batch = _sc_tile(tp_flat, type_index,
                                                   batch)
    sites_params = jnp.transpose(params_flat.reshape(PARAM_DIM, N_ATOMS))
    one = (jnp.minimum(type_index[0], 0) + 1).astype(jnp.float32)
    sites_pos = pos * one
    return (sites_pos, sites_params, sites_batch, sites_mol)
